# baseline (speedup 1.0000x reference)
import jax
import jax.numpy as jnp
from jax.experimental import pallas as pl
from jax.experimental.pallas import tpu as pltpu

_CHUNK = 512
_NBUF = 4


def kernel(x, emb_weight):
    sl = x.shape[1]
    dim = emb_weight.shape[1]
    n = sl // _CHUNK

    def body(w_ref, o_ref, buf, in_sems, out_sems):
        def in_copy(i):
            return pltpu.make_async_copy(
                w_ref.at[pl.ds(i * _CHUNK, _CHUNK)],
                buf.at[i % _NBUF],
                in_sems.at[i % _NBUF],
            )

        def out_copy(i):
            return pltpu.make_async_copy(
                buf.at[i % _NBUF],
                o_ref.at[pl.ds(i * _CHUNK, _CHUNK)],
                out_sems.at[i % _NBUF],
            )

        for i in range(min(_NBUF, n)):
            in_copy(i).start()
        for i in range(n):
            in_copy(i).wait()
            out_copy(i).start()
            j = i + _NBUF
            if j < n:
                out_copy(i).wait()
                in_copy(j).start()
        for i in range(max(0, n - _NBUF), n):
            out_copy(i).wait()

    return pl.pallas_call(
        body,
        in_specs=[pl.BlockSpec(memory_space=pl.ANY)],
        out_specs=pl.BlockSpec(memory_space=pl.ANY),
        out_shape=jax.ShapeDtypeStruct((sl, dim), emb_weight.dtype),
        scratch_shapes=[
            pltpu.VMEM((_NBUF, _CHUNK, dim), jnp.float32),
            pltpu.SemaphoreType.DMA((_NBUF,)),
            pltpu.SemaphoreType.DMA((_NBUF,)),
        ],
    )(emb_weight)


# speedup vs baseline: 1.1261x; 1.1261x over previous
import jax
import jax.numpy as jnp
from jax.experimental import pallas as pl
from jax.experimental.pallas import tpu as pltpu

_CHUNK = 2048
_NBUF = 2


def kernel(x, emb_weight):
    sl = x.shape[1]
    dim = emb_weight.shape[1]
    n = sl // _CHUNK

    def body(w_ref, o_ref, buf, in_sems, out_sems):
        def in_copy(i):
            return pltpu.make_async_copy(
                w_ref.at[pl.ds(i * _CHUNK, _CHUNK)],
                buf.at[i % _NBUF],
                in_sems.at[i % _NBUF],
            )

        def out_copy(i):
            return pltpu.make_async_copy(
                buf.at[i % _NBUF],
                o_ref.at[pl.ds(i * _CHUNK, _CHUNK)],
                out_sems.at[i % _NBUF],
            )

        for i in range(min(_NBUF, n)):
            in_copy(i).start()
        for i in range(n):
            in_copy(i).wait()
            out_copy(i).start()
            j = i + _NBUF
            if j < n:
                out_copy(i).wait()
                in_copy(j).start()
        for i in range(max(0, n - _NBUF), n):
            out_copy(i).wait()

    return pl.pallas_call(
        body,
        in_specs=[pl.BlockSpec(memory_space=pl.ANY)],
        out_specs=pl.BlockSpec(memory_space=pl.ANY),
        out_shape=jax.ShapeDtypeStruct((sl, dim), emb_weight.dtype),
        scratch_shapes=[
            pltpu.VMEM((_NBUF, _CHUNK, dim), jnp.float32),
            pltpu.SemaphoreType.DMA((_NBUF,)),
            pltpu.SemaphoreType.DMA((_NBUF,)),
        ],
    )(emb_weight)


# speedup vs baseline: 1.1871x; 1.0542x over previous
import jax
import jax.numpy as jnp
from jax.experimental import pallas as pl
from jax.experimental.pallas import tpu as pltpu

_CHUNK = 1024
_NBUF = 6


def kernel(x, emb_weight):
    sl = x.shape[1]
    dim = emb_weight.shape[1]
    n = sl // _CHUNK

    def body(w_ref, o_ref, buf, in_sems, out_sems):
        def in_copy(i):
            return pltpu.make_async_copy(
                w_ref.at[pl.ds(i * _CHUNK, _CHUNK)],
                buf.at[i % _NBUF],
                in_sems.at[i % _NBUF],
            )

        def out_copy(i):
            return pltpu.make_async_copy(
                buf.at[i % _NBUF],
                o_ref.at[pl.ds(i * _CHUNK, _CHUNK)],
                out_sems.at[i % _NBUF],
            )

        for i in range(min(_NBUF, n)):
            in_copy(i).start()
        for i in range(n):
            in_copy(i).wait()
            out_copy(i).start()
            j = i + _NBUF
            if j < n:
                out_copy(i).wait()
                in_copy(j).start()
        for i in range(max(0, n - _NBUF), n):
            out_copy(i).wait()

    return pl.pallas_call(
        body,
        in_specs=[pl.BlockSpec(memory_space=pl.ANY)],
        out_specs=pl.BlockSpec(memory_space=pl.ANY),
        out_shape=jax.ShapeDtypeStruct((sl, dim), emb_weight.dtype),
        scratch_shapes=[
            pltpu.VMEM((_NBUF, _CHUNK, dim), jnp.float32),
            pltpu.SemaphoreType.DMA((_NBUF,)),
            pltpu.SemaphoreType.DMA((_NBUF,)),
        ],
    )(emb_weight)


# speedup vs baseline: 1.1971x; 1.0084x over previous
import jax
import jax.numpy as jnp
from jax.experimental import pallas as pl
from jax.experimental.pallas import tpu as pltpu

_CHUNK = 1024
_NBUF = 5


def kernel(x, emb_weight):
    sl = x.shape[1]
    dim = emb_weight.shape[1]
    n = sl // _CHUNK

    def body(w_ref, o_ref, buf, in_sems, out_sems):
        def in_copy(i):
            return pltpu.make_async_copy(
                w_ref.at[pl.ds(i * _CHUNK, _CHUNK)],
                buf.at[i % _NBUF],
                in_sems.at[i % _NBUF],
            )

        def out_copy(i):
            return pltpu.make_async_copy(
                buf.at[i % _NBUF],
                o_ref.at[pl.ds(i * _CHUNK, _CHUNK)],
                out_sems.at[i % _NBUF],
            )

        for i in range(min(_NBUF, n)):
            in_copy(i).start()
        for i in range(n):
            in_copy(i).wait()
            out_copy(i).start()
            j = i + _NBUF
            if j < n:
                out_copy(i).wait()
                in_copy(j).start()
        for i in range(max(0, n - _NBUF), n):
            out_copy(i).wait()

    return pl.pallas_call(
        body,
        in_specs=[pl.BlockSpec(memory_space=pl.ANY)],
        out_specs=pl.BlockSpec(memory_space=pl.ANY),
        out_shape=jax.ShapeDtypeStruct((sl, dim), emb_weight.dtype),
        scratch_shapes=[
            pltpu.VMEM((_NBUF, _CHUNK, dim), jnp.float32),
            pltpu.SemaphoreType.DMA((_NBUF,)),
            pltpu.SemaphoreType.DMA((_NBUF,)),
        ],
    )(emb_weight)
